# baseline (speedup 1.0000x reference)
import jax
import jax.numpy as jnp
from jax.experimental import pallas as pl
from jax.experimental.pallas import tpu as pltpu

_E, _D, _F, _T = 8, 768, 2048, 2048
_FC = 1024
_NF = _F // _FC


def _moe_body(x_ref, w1_ref, b1_ref, w2_ref, b2a_ref, wr_ref, br_ref,
              out_ref, w_scr, xb_scr):
    e = pl.program_id(0)
    f = pl.program_id(1)
    step = e * _NF + f

    @pl.when(step == 0)
    def _init():
        logits = jnp.dot(x_ref[...], wr_ref[...],
                         preferred_element_type=jnp.float32)
        w = jax.nn.softmax(logits + br_ref[...], axis=-1)
        w_scr[...] = w
        xb_scr[...] = x_ref[...].astype(jnp.bfloat16)
        out_ref[...] = jnp.dot(w, b2a_ref[...],
                               preferred_element_type=jnp.float32)

    w = w_scr[...]
    lane = jax.lax.broadcasted_iota(jnp.int32, w.shape, 1)
    w_e = jnp.sum(jnp.where(lane == e, w, 0.0), axis=1, keepdims=True)

    h = jnp.dot(xb_scr[...], w1_ref[0].astype(jnp.bfloat16),
                preferred_element_type=jnp.float32)
    h = h + b1_ref[0]
    h = 0.5 * h * (1.0 + jax.lax.erf(h * 0.7071067811865476))
    o = jnp.dot(h.astype(jnp.bfloat16), w2_ref[0].astype(jnp.bfloat16),
                preferred_element_type=jnp.float32)
    out_ref[...] += w_e * o


def kernel(x, W1, b1, W2, b2, Wr, br):
    grid = (_E, _NF)
    out = pl.pallas_call(
        _moe_body,
        grid=grid,
        in_specs=[
            pl.BlockSpec((_T, _D), lambda e, f: (0, 0)),
            pl.BlockSpec((1, _D, _FC), lambda e, f: (e, 0, f)),
            pl.BlockSpec((1, 1, _FC), lambda e, f: (e, 0, f)),
            pl.BlockSpec((1, _FC, _D), lambda e, f: (e, f, 0)),
            pl.BlockSpec((_E, _D), lambda e, f: (0, 0)),
            pl.BlockSpec((_D, _E), lambda e, f: (0, 0)),
            pl.BlockSpec((1, _E), lambda e, f: (0, 0)),
        ],
        out_specs=pl.BlockSpec((_T, _D), lambda e, f: (0, 0)),
        out_shape=jax.ShapeDtypeStruct((_T, _D), jnp.float32),
        scratch_shapes=[
            pltpu.VMEM((_T, _E), jnp.float32),
            pltpu.VMEM((_T, _D), jnp.bfloat16),
        ],
        compiler_params=pltpu.CompilerParams(
            dimension_semantics=("arbitrary", "arbitrary"),
            vmem_limit_bytes=112 * 1024 * 1024,
        ),
    )(x, W1, b1.reshape(_E, 1, _F), W2, b2, Wr, br.reshape(1, _E))
    return out


# speedup vs baseline: 1.0171x; 1.0171x over previous
import jax
import jax.numpy as jnp
from jax.experimental import pallas as pl
from jax.experimental.pallas import tpu as pltpu

_E, _D, _F, _T = 8, 768, 2048, 2048


def _moe_body(x_ref, w1_ref, b1_ref, w2_ref, b2a_ref, wr_ref, br_ref,
              out_ref, w_scr, xb_scr):
    e = pl.program_id(0)

    @pl.when(e == 0)
    def _init():
        logits = jnp.dot(x_ref[...], wr_ref[...],
                         preferred_element_type=jnp.float32)
        w = jax.nn.softmax(logits + br_ref[...], axis=-1)
        w_scr[...] = w
        xb_scr[...] = x_ref[...].astype(jnp.bfloat16)
        out_ref[...] = jnp.dot(w, b2a_ref[...],
                               preferred_element_type=jnp.float32)

    w = w_scr[...]
    lane = jax.lax.broadcasted_iota(jnp.int32, w.shape, 1)
    w_e = jnp.sum(jnp.where(lane == e, w, 0.0), axis=1, keepdims=True)

    h = jnp.dot(xb_scr[...], w1_ref[0].astype(jnp.bfloat16),
                preferred_element_type=jnp.float32)
    h = h + b1_ref[0]
    h = 0.5 * h * (1.0 + jax.lax.erf(h * 0.7071067811865476))
    o = jnp.dot(h.astype(jnp.bfloat16), w2_ref[0].astype(jnp.bfloat16),
                preferred_element_type=jnp.float32)
    out_ref[...] += w_e * o


def kernel(x, W1, b1, W2, b2, Wr, br):
    grid = (_E,)
    out = pl.pallas_call(
        _moe_body,
        grid=grid,
        in_specs=[
            pl.BlockSpec((_T, _D), lambda e: (0, 0)),
            pl.BlockSpec((1, _D, _F), lambda e: (e, 0, 0)),
            pl.BlockSpec((1, 1, _F), lambda e: (e, 0, 0)),
            pl.BlockSpec((1, _F, _D), lambda e: (e, 0, 0)),
            pl.BlockSpec((_E, _D), lambda e: (0, 0)),
            pl.BlockSpec((_D, _E), lambda e: (0, 0)),
            pl.BlockSpec((1, _E), lambda e: (0, 0)),
        ],
        out_specs=pl.BlockSpec((_T, _D), lambda e: (0, 0)),
        out_shape=jax.ShapeDtypeStruct((_T, _D), jnp.float32),
        scratch_shapes=[
            pltpu.VMEM((_T, _E), jnp.float32),
            pltpu.VMEM((_T, _D), jnp.bfloat16),
        ],
        compiler_params=pltpu.CompilerParams(
            dimension_semantics=("arbitrary",),
            vmem_limit_bytes=112 * 1024 * 1024,
        ),
    )(x, W1, b1.reshape(_E, 1, _F), W2, b2, Wr, br.reshape(1, _E))
    return out
